# baseline (speedup 1.0000x reference)
import functools

import jax
import jax.numpy as jnp
from jax import lax
from jax.experimental import pallas as pl
from jax.experimental.pallas import tpu as pltpu
from jax.experimental.pallas import tpu_sc as plsc

PAD = 1
L = 16


def _sc_kernel(inp_flat, weights, bsz, seq, d):
    NC, NS = 2, 16
    NW = NC * NS
    n = bsz * seq
    sl = n // NW
    wpr = NW // bsz
    G = 32
    ng = sl // G
    vpc = G // L

    mesh = plsc.VectorSubcoreMesh(core_axis_name="c", subcore_axis_name="s")

    @functools.partial(
        pl.kernel,
        out_type=jax.ShapeDtypeStruct((n, d), jnp.float32),
        mesh=mesh,
        scratch_types=[
            pltpu.VMEM((seq,), jnp.int32),
            pltpu.VMEM((sl,), jnp.int32),
            pltpu.VMEM((3 * L,), jnp.int32),
            pltpu.VMEM((3, G, d), jnp.float32),
            pltpu.SemaphoreType.DMA,
            pltpu.SemaphoreType.DMA,
            pltpu.SemaphoreType.DMA,
            pltpu.SemaphoreType.DMA,
            pltpu.SemaphoreType.DMA,
            pltpu.SemaphoreType.DMA,
        ],
    )
    def k(inp_hbm, tab_hbm, out_hbm, row_v, idx_v, sh_v, rows_v,
          sg0, sg1, sg2, so0, so1, so2):
        wid = lax.axis_index("s") * NC + lax.axis_index("c")
        b = wid // wpr
        c = wid % wpr
        off = c * sl
        base = wid * sl

        pltpu.sync_copy(inp_hbm.at[pl.ds(b * seq, seq)], row_v)

        zero = jnp.zeros((L,), jnp.int32)
        sh_v[pl.ds(0, L)] = zero
        sh_v[pl.ds(2 * L, L)] = zero

        def shift_scans(x):
            p = x
            for kk in (1, 2, 4, 8):
                sh_v[pl.ds(L, L)] = p
                p = p + sh_v[pl.ds(L - kk, L)]
            s = x
            for kk in (1, 2, 4, 8):
                sh_v[pl.ds(L, L)] = s
                s = s + sh_v[pl.ds(L + kk, L)]
            return p, s

        def pc_body(i, acc):
            v = row_v[pl.ds(i * L, L)]
            return acc + jnp.where(v != PAD, 1, 0)

        acc = lax.fori_loop(0, off // L, pc_body, zero)
        p0, s0 = shift_scans(acc)
        carry0 = p0 + s0 - acc

        def chunk_positions(g, carry):
            for t in range(vpc):
                v = row_v[pl.ds(off + g * G + t * L, L)]
                m = jnp.where(v != PAD, 1, 0)
                p, s = shift_scans(m)
                idx_v[pl.ds(g * G + t * L, L)] = (carry + p) * m + PAD
                carry = carry + (p + s - m)
            return carry

        r = [rows_v.at[0], rows_v.at[1], rows_v.at[2]]
        sg = [sg0, sg1, sg2]
        so = [so0, so1, so2]

        def gath(g, j):
            pltpu.async_copy(tab_hbm.at[idx_v.at[pl.ds(g * G, G)]], r[j], sg[j])

        def outw(g, j):
            pltpu.async_copy(r[j], out_hbm.at[pl.ds(base + g * G, G)], so[j])

        def wait_g(j):
            pltpu.make_async_copy(tab_hbm.at[pl.ds(0, G)], r[j], sg[j]).wait()

        def wait_o(j):
            pltpu.make_async_copy(r[j], out_hbm.at[pl.ds(base, G)], so[j]).wait()

        carry = carry0
        for j in range(3):
            carry = chunk_positions(j, carry)
        for j in range(3):
            gath(j, j)

        def body(h, carry):
            g = 3 * h
            for j in range(3):
                carry = chunk_positions(g + 3 + j, carry)
            for j in range(3):
                wait_g(j)
                outw(g + j, j)
                wait_o(j)
                gath(g + j + 3, j)
            return carry

        carry = lax.fori_loop(0, (ng - 5) // 3, body, carry)

        gtail = ng - 5
        carry = chunk_positions(gtail + 3, carry)
        carry = chunk_positions(gtail + 4, carry)
        for j in range(3):
            wait_g(j)
            outw(gtail + j, j)
            if j < 2:
                wait_o(j)
                gath(gtail + j + 3, j)
        for j in range(2):
            wait_g(j)
            outw(ng - 2 + j, j)
        for j in range(3):
            wait_o(j)

    return k(inp_flat, weights)


def kernel(input, weights):
    bsz, seq = input.shape
    nrows, d = weights.shape
    out = _sc_kernel(input.reshape(bsz * seq), weights, bsz, seq, d)
    return lax.stop_gradient(out.reshape(bsz, seq, d))


# speedup vs baseline: 1.0071x; 1.0071x over previous
import functools

import jax
import jax.numpy as jnp
from jax import lax
from jax.experimental import pallas as pl
from jax.experimental.pallas import tpu as pltpu
from jax.experimental.pallas import tpu_sc as plsc

PAD = 1
L = 16


def _sc_kernel(inp_flat, weights, bsz, seq, d):
    NC, NS = 2, 16
    NW = NC * NS
    n = bsz * seq
    sl = n // NW
    wpr = NW // bsz
    G = 32
    ng = sl // G
    vpc = G // L

    mesh = plsc.VectorSubcoreMesh(core_axis_name="c", subcore_axis_name="s")

    @functools.partial(
        pl.kernel,
        out_type=jax.ShapeDtypeStruct((n, d), jnp.float32),
        mesh=mesh,
        scratch_types=[
            pltpu.VMEM((seq,), jnp.int32),
            pltpu.VMEM((sl,), jnp.int32),
            pltpu.VMEM((3 * L,), jnp.int32),
            pltpu.VMEM((3, G, d), jnp.float32),
            pltpu.SemaphoreType.DMA,
            pltpu.SemaphoreType.DMA,
            pltpu.SemaphoreType.DMA,
            pltpu.SemaphoreType.DMA,
            pltpu.SemaphoreType.DMA,
            pltpu.SemaphoreType.DMA,
        ],
    )
    def k(inp_hbm, tab_hbm, out_hbm, row_v, idx_v, sh_v, rows_v,
          sg0, sg1, sg2, so0, so1, so2):
        wid = lax.axis_index("s") * NC + lax.axis_index("c")
        b = wid // wpr
        c = wid % wpr
        off = c * sl
        base = wid * sl

        pltpu.sync_copy(inp_hbm.at[pl.ds(b * seq, seq)], row_v)

        zero = jnp.zeros((L,), jnp.int32)
        sh_v[pl.ds(0, L)] = zero
        sh_v[pl.ds(2 * L, L)] = zero

        def shift_scans(x):
            p = x
            for kk in (1, 2, 4, 8):
                sh_v[pl.ds(L, L)] = p
                p = p + sh_v[pl.ds(L - kk, L)]
            s = x
            for kk in (1, 2, 4, 8):
                sh_v[pl.ds(L, L)] = s
                s = s + sh_v[pl.ds(L + kk, L)]
            return p, s

        def pc_body(i, acc):
            for u in range(8):
                v = row_v[pl.ds(i * 8 * L + u * L, L)]
                acc = acc + jnp.where(v != PAD, 1, 0)
            return acc

        acc = lax.fori_loop(0, off // (8 * L), pc_body, zero)
        p0, s0 = shift_scans(acc)
        carry0 = p0 + s0 - acc

        def chunk_positions(g, carry):
            for t in range(vpc):
                v = row_v[pl.ds(off + g * G + t * L, L)]
                m = jnp.where(v != PAD, 1, 0)
                p, s = shift_scans(m)
                idx_v[pl.ds(g * G + t * L, L)] = (carry + p) * m + PAD
                carry = carry + (p + s - m)
            return carry

        r = [rows_v.at[0], rows_v.at[1], rows_v.at[2]]
        sg = [sg0, sg1, sg2]
        so = [so0, so1, so2]

        def gath(g, j):
            pltpu.async_copy(tab_hbm.at[idx_v.at[pl.ds(g * G, G)]], r[j], sg[j])

        def outw(g, j):
            pltpu.async_copy(r[j], out_hbm.at[pl.ds(base + g * G, G)], so[j])

        def wait_g(j):
            pltpu.make_async_copy(tab_hbm.at[pl.ds(0, G)], r[j], sg[j]).wait()

        def wait_o(j):
            pltpu.make_async_copy(r[j], out_hbm.at[pl.ds(base, G)], so[j]).wait()

        carry = carry0
        for j in range(3):
            carry = chunk_positions(j, carry)
        for j in range(3):
            gath(j, j)

        def body(h, carry):
            g = 3 * h
            for j in range(3):
                carry = chunk_positions(g + 3 + j, carry)
            for j in range(3):
                wait_g(j)
                outw(g + j, j)
                wait_o(j)
                gath(g + j + 3, j)
            return carry

        carry = lax.fori_loop(0, (ng - 5) // 3, body, carry)

        gtail = ng - 5
        carry = chunk_positions(gtail + 3, carry)
        carry = chunk_positions(gtail + 4, carry)
        for j in range(3):
            wait_g(j)
            outw(gtail + j, j)
            if j < 2:
                wait_o(j)
                gath(gtail + j + 3, j)
        for j in range(2):
            wait_g(j)
            outw(ng - 2 + j, j)
        for j in range(3):
            wait_o(j)

    return k(inp_flat, weights)


def kernel(input, weights):
    bsz, seq = input.shape
    nrows, d = weights.shape
    out = _sc_kernel(input.reshape(bsz * seq), weights, bsz, seq, d)
    return lax.stop_gradient(out.reshape(bsz, seq, d))
